# Initial kernel scaffold; baseline (speedup 1.0000x reference)
import jax
import jax.numpy as jnp
from jax.experimental import pallas as pl


def kernel(x, params, edge_index, batch):
    raise NotImplementedError("write your pallas kernel here")


# speedup vs baseline: 1.0001x; 1.0001x over previous
import functools
import math

import jax
import jax.numpy as jnp
from jax.experimental import pallas as pl
from jax.experimental.pallas import tpu as pltpu

NUM_NODES = 50000
NUM_EDGES = 800000
NUM_GRAPHS = 32
NUM_CLASSES = 40


def _elu(x):
    return jnp.where(x > 0, x, jnp.exp(x) - 1.0)


def _leaky_relu(x, s=0.2):
    return jnp.where(x > 0, x, s * x)


def _head_body(pooled_ref, l1w_ref, l1b_ref, l2w_ref, l2b_ref, l3w_ref,
               l3b_ref, out_ref):
    h = pooled_ref[...]
    h = _elu(jnp.dot(h, l1w_ref[...], preferred_element_type=jnp.float32)
             + l1b_ref[...])
    h = _elu(jnp.dot(h, l2w_ref[...], preferred_element_type=jnp.float32)
             + l2b_ref[...])
    h = jnp.dot(h, l3w_ref[...], preferred_element_type=jnp.float32) + l3b_ref[...]
    m = jnp.max(h, axis=1, keepdims=True)
    s = jnp.log(jnp.sum(jnp.exp(h - m), axis=1, keepdims=True))
    out_ref[...] = h - m - s


def _head(pooled, p):
    return pl.pallas_call(
        _head_body,
        out_shape=jax.ShapeDtypeStruct((NUM_GRAPHS, NUM_CLASSES), jnp.float32),
    )(pooled, p['L1w'], p['L1b'][None, :], p['L2w'], p['L2b'][None, :],
      p['L3w'], p['L3b'][None, :])


def _gat_conv(x, row, col, W, a_src, a_dst, b, heads, out_ch, concat, num_nodes):
    sl = jnp.arange(num_nodes, dtype=jnp.int32)
    row = jnp.concatenate([row, sl])
    col = jnp.concatenate([col, sl])
    h = (x @ W).reshape(num_nodes, heads, out_ch)
    a_s = (h * a_src[None]).sum(-1)
    a_d = (h * a_dst[None]).sum(-1)
    alpha = _leaky_relu(a_s[row] + a_d[col])
    amax = jax.ops.segment_max(alpha, col, num_segments=num_nodes)
    amax = jnp.where(jnp.isfinite(amax), amax, 0.0)
    ae = jnp.exp(alpha - amax[col])
    asum = jax.ops.segment_sum(ae, col, num_segments=num_nodes)
    att = ae / (asum[col] + 1e-16)
    out = jax.ops.segment_sum(h[row] * att[..., None], col, num_segments=num_nodes)
    if concat:
        out = out.reshape(num_nodes, heads * out_ch)
    else:
        out = out.mean(axis=1)
    return out + b


def _instance_norm(x, batch, gamma, beta, num_graphs, eps=1e-5):
    ones = jnp.ones((x.shape[0],), x.dtype)
    cnt = jnp.maximum(jax.ops.segment_sum(ones, batch, num_segments=num_graphs), 1.0)[:, None]
    mean = jax.ops.segment_sum(x, batch, num_segments=num_graphs) / cnt
    var = jax.ops.segment_sum(x * x, batch, num_segments=num_graphs) / cnt - mean ** 2
    xh = (x - mean[batch]) * jax.lax.rsqrt(var[batch] + eps)
    return xh * gamma + beta


def _topk_perm(score, batch_key, valid, num_nodes):
    idx1 = jnp.argsort(-score)
    order = idx1[jnp.argsort(batch_key[idx1])]
    bs = batch_key[order]
    total = jax.ops.segment_sum(jnp.ones((num_nodes,), jnp.int32), batch_key, num_segments=NUM_GRAPHS + 1)
    starts = jnp.concatenate([jnp.zeros((1,), jnp.int32), jnp.cumsum(total)[:-1]])
    rank = jnp.arange(num_nodes, dtype=jnp.int32) - starts[bs]
    cnt = jax.ops.segment_sum(valid.astype(jnp.int32), batch_key, num_segments=NUM_GRAPHS + 1)
    k = jnp.maximum((3 * cnt + 9) // 10, 1)
    keep = valid[order] & (rank < k[bs])
    pos = jnp.cumsum(keep.astype(jnp.int32)) - 1
    nsel = keep.astype(jnp.int32).sum()
    src = jnp.zeros((num_nodes,), jnp.int32).at[jnp.where(keep, pos, num_nodes)].set(
        order.astype(jnp.int32), mode='drop')
    newidx = jnp.full((num_nodes,), -1, jnp.int32).at[order].set(jnp.where(keep, pos, -1))
    valid_new = jnp.arange(num_nodes, dtype=jnp.int32) < nsel
    return src, newidx, valid_new


def _filter_adj(row, col, newidx, pad_node, num_edges):
    nr = newidx[row]
    nc = newidx[col]
    keep = (nr >= 0) & (nc >= 0)
    pos = jnp.cumsum(keep.astype(jnp.int32)) - 1
    tgt = jnp.where(keep, pos, num_edges)
    row_new = jnp.full((num_edges,), pad_node, jnp.int32).at[tgt].set(nr, mode='drop')
    col_new = jnp.full((num_edges,), pad_node, jnp.int32).at[tgt].set(nc, mode='drop')
    return row_new, col_new


def kernel(x, params, edge_index, batch):
    p = params
    N0 = x.shape[0]
    pad_node = N0 - 1
    row0 = jnp.asarray(edge_index[0], jnp.int32)
    col0 = jnp.asarray(edge_index[1], jnp.int32)
    batch0 = jnp.asarray(batch, jnp.int32)
    h = _elu(_gat_conv(x, row0, col0, p['W1'], p['as1'], p['ad1'], p['b1'], 2, 32, True, N0))
    h = _instance_norm(h, batch0, p['g1'], p['be1'], NUM_GRAPHS)
    score1 = jax.nn.sigmoid((h @ p['pw1']) / (jnp.linalg.norm(p['pw1']) + 1e-16))
    valid0 = jnp.ones((N0,), bool)
    src1, newidx1, valid1 = _topk_perm(score1, batch0, valid0, N0)
    row1, col1 = _filter_adj(row0, col0, newidx1, pad_node, NUM_EDGES)
    h = jnp.where(valid1[:, None], h[src1] * score1[src1][:, None], 0.0)
    batch1 = jnp.where(valid1, batch0[src1], NUM_GRAPHS).astype(jnp.int32)
    h = _elu(_gat_conv(h, row1, col1, p['W2'], p['as2'], p['ad2'], p['b2'], 2, 128, True, N0))
    h = _instance_norm(h, batch1, p['g2'], p['be2'], NUM_GRAPHS)
    h = jnp.where(valid1[:, None], h, 0.0)
    score2 = jax.nn.sigmoid((h @ p['pw2']) / (jnp.linalg.norm(p['pw2']) + 1e-16))
    src2, newidx2, valid2 = _topk_perm(score2, batch1, valid1, N0)
    row2, col2 = _filter_adj(row1, col1, newidx2, pad_node, NUM_EDGES)
    h = jnp.where(valid2[:, None], h[src2] * score2[src2][:, None], 0.0)
    batch2 = jnp.where(valid2, batch1[src2], NUM_GRAPHS).astype(jnp.int32)
    h = _elu(_gat_conv(h, row2, col2, p['W3'], p['as3'], p['ad3'], p['b3'], 2, 512, False, N0))
    h = _instance_norm(h, batch2, p['g3'], p['be3'], NUM_GRAPHS)
    h = jnp.where(valid2[:, None], h, 0.0)
    gate = _elu(h @ p['Gw1'] + p['Gb1'])
    gate = _elu(gate @ p['Gw2'] + p['Gb2'])
    gate = (gate @ p['Gw3'] + p['Gb3'])[:, 0]
    feat = _elu(h @ p['Aw'] + p['Ab'])
    gmax = jax.ops.segment_max(gate, batch2, num_segments=NUM_GRAPHS)
    gmax = jnp.where(jnp.isfinite(gmax), gmax, 0.0)
    ge = jnp.exp(gate - gmax[batch2])
    gs = jax.ops.segment_sum(ge, batch2, num_segments=NUM_GRAPHS)
    w = ge / (gs[batch2] + 1e-16)
    pooled = jax.ops.segment_sum(feat * w[:, None], batch2, num_segments=NUM_GRAPHS)
    return _head(pooled, p)


# speedup vs baseline: 9.4306x; 9.4299x over previous
import functools
import math

import jax
import jax.numpy as jnp
from jax import lax
from jax.experimental import pallas as pl
from jax.experimental.pallas import tpu as pltpu
from jax.experimental.pallas import tpu_sc as plsc

NUM_NODES = 50000
NUM_EDGES = 800000
NUM_GRAPHS = 32
NUM_CLASSES = 40

CH = 128
NT = 16
FB = 32
STRIPE = 3128
HEADS = 2


def _elu(x):
    return jnp.where(x > 0, x, jnp.exp(x) - 1.0)


def _leaky_relu(x, s=0.2):
    return jnp.where(x > 0, x, s * x)


def _head_body(pooled_ref, l1w_ref, l1b_ref, l2w_ref, l2b_ref, l3w_ref,
               l3b_ref, out_ref):
    h = pooled_ref[...]
    h = _elu(jnp.dot(h, l1w_ref[...], preferred_element_type=jnp.float32)
             + l1b_ref[...])
    h = _elu(jnp.dot(h, l2w_ref[...], preferred_element_type=jnp.float32)
             + l2b_ref[...])
    h = jnp.dot(h, l3w_ref[...], preferred_element_type=jnp.float32) + l3b_ref[...]
    m = jnp.max(h, axis=1, keepdims=True)
    s = jnp.log(jnp.sum(jnp.exp(h - m), axis=1, keepdims=True))
    out_ref[...] = h - m - s


def _head(pooled, p):
    return pl.pallas_call(
        _head_body,
        out_shape=jax.ShapeDtypeStruct((NUM_GRAPHS, NUM_CLASSES), jnp.float32),
    )(pooled, p['L1w'], p['L1b'][None, :], p['L2w'], p['L2b'][None, :],
      p['L3w'], p['L3b'][None, :])


NP = 50048


def _sc_mesh():
    return plsc.VectorSubcoreMesh(core_axis_name="c", subcore_axis_name="s")


@functools.cache
def _edge_w_kernel(E):
    NCHUNK = E // CH
    NIT = (NCHUNK + NT - 1) // NT

    @functools.partial(
        pl.kernel,
        mesh=_sc_mesh(),
        out_type=[jax.ShapeDtypeStruct((HEADS * E,), jnp.float32),
                  jax.ShapeDtypeStruct((HEADS * NP,), jnp.float32)],
        scratch_types=[
            pltpu.VMEM((1, CH), jnp.int32),
            pltpu.VMEM((1, CH), jnp.int32),
            pltpu.VMEM((1, CH), jnp.int32),
            pltpu.VMEM((1, CH), jnp.int32),
            pltpu.VMEM((1, CH), jnp.float32),
            pltpu.VMEM((1, CH), jnp.float32),
            pltpu.VMEM((1, CH), jnp.float32),
            pltpu.VMEM((16,), jnp.int32),
            pltpu.VMEM((3200,), jnp.float32),
            pltpu.VMEM_SHARED((NP,), jnp.float32),
            pltpu.SemaphoreType.DMA,
        ],
    )
    def kern(asf, adf, row_hbm, col_hbm, nk_hbm, w_hbm, asum_hbm,
             rowi, coli, gr, gc, asb, adb, wbuf, nk_v, zbuf, asum_sp, sem):
        head = lax.axis_index("c")
        sid = lax.axis_index("s")
        pltpu.sync_copy(nk_hbm, nk_v)
        nkvec = nk_v[...]
        s_nk = nkvec[0]
        zero16 = jnp.zeros((16,), jnp.float32)

        def zbody(i, c):
            zbuf[pl.ds(i * 16, 16)] = zero16
            return c
        lax.fori_loop(0, 200, zbody, 0)
        pltpu.sync_copy(zbuf.at[pl.ds(0, STRIPE)],
                        asum_sp.at[pl.ds(sid * STRIPE, STRIPE)])
        plsc.subcore_barrier()

        def body(it, c):
            cid = it * NT + sid
            base = cid * CH

            @pl.when(jnp.logical_and(cid < NCHUNK, base < s_nk))
            def _():
                pltpu.sync_copy(row_hbm.at[pl.ds(base, CH)], rowi.at[0])
                pltpu.sync_copy(col_hbm.at[pl.ds(base, CH)], coli.at[0])
                off = head * NP
                for j in range(CH // 16):
                    gr[0, pl.ds(j * 16, 16)] = rowi[0, pl.ds(j * 16, 16)] + off
                    gc[0, pl.ds(j * 16, 16)] = coli[0, pl.ds(j * 16, 16)] + off
                cp1 = pltpu.async_copy(asf.at[gr.at[0]], asb.at[0], sem)
                cp2 = pltpu.async_copy(adf.at[gc.at[0]], adb.at[0], sem)
                cp1.wait()
                cp2.wait()
                for j in range(CH // 16):
                    a = asb[0, pl.ds(j * 16, 16)] + adb[0, pl.ds(j * 16, 16)]
                    a = jnp.where(a > 0, a, 0.2 * a)
                    w = jnp.exp(a)
                    lane = base + j * 16 + lax.iota(jnp.int32, 16)
                    w = jnp.where(lane < nkvec, w, 0.0)
                    wbuf[0, pl.ds(j * 16, 16)] = w
                pltpu.sync_copy(wbuf.at[0], asum_sp.at[coli.at[0]], add=True)
                pltpu.sync_copy(wbuf.at[0], w_hbm.at[pl.ds(head * E + base, CH)])
            return c
        lax.fori_loop(0, NIT, body, 0)
        plsc.subcore_barrier()
        pltpu.sync_copy(asum_sp.at[pl.ds(sid * STRIPE, STRIPE)],
                        zbuf.at[pl.ds(0, STRIPE)])
        pltpu.sync_copy(zbuf.at[pl.ds(0, STRIPE)],
                        asum_hbm.at[pl.ds(head * NP + sid * STRIPE, STRIPE)])

    return kern


@functools.cache
def _edge_agg_kernel(E, S):
    NCHUNK = E // CH
    NIT = (NCHUNK + NT - 1) // NT
    ZR = 136

    @functools.partial(
        pl.kernel,
        mesh=_sc_mesh(),
        compiler_params=pltpu.CompilerParams(use_tc_tiling_on_sc=False),
        out_type=jax.ShapeDtypeStruct((HEADS * S * NP, FB), jnp.float32),
        scratch_types=[
            pltpu.VMEM((1, CH), jnp.int32),
            pltpu.VMEM((1, CH), jnp.int32),
            pltpu.VMEM((1, CH), jnp.int32),
            pltpu.VMEM((1, CH), jnp.float32),
            pltpu.VMEM((CH, FB), jnp.float32),
            pltpu.VMEM((16,), jnp.int32),
            pltpu.VMEM((ZR, FB), jnp.float32),
            pltpu.VMEM_SHARED((NP, FB), jnp.float32),
            pltpu.SemaphoreType.DMA,
        ],
    )
    def kern(hblk, row_hbm, col_hbm, w_hbm, nk_hbm, out_hbm,
             rowi, coli, gidx, wbuf, rows, nk_v, zrow, acc_sp, sem):
        head = lax.axis_index("c")
        sid = lax.axis_index("s")
        pltpu.sync_copy(nk_hbm, nk_v)
        s_nk = nk_v[...][0]
        zero16 = jnp.zeros((16,), jnp.float32)
        for r in range(ZR):
            for cpart in range(FB // 16):
                zrow[r, pl.ds(cpart * 16, 16)] = zero16

        def subpass(s, carry):
            tb = head * S + s

            def zcopy(kk, c):
                pltpu.sync_copy(
                    zrow, acc_sp.at[pl.ds(sid * STRIPE + kk * ZR, ZR)])
                return c
            lax.fori_loop(0, STRIPE // ZR, zcopy, 0)
            plsc.subcore_barrier()

            def body(it, c):
                cid = it * NT + sid
                base = cid * CH

                @pl.when(jnp.logical_and(cid < NCHUNK, base < s_nk))
                def _():
                    pltpu.sync_copy(row_hbm.at[pl.ds(base, CH)], rowi.at[0])
                    pltpu.sync_copy(col_hbm.at[pl.ds(base, CH)], coli.at[0])
                    pltpu.sync_copy(w_hbm.at[pl.ds(head * E + base, CH)],
                                    wbuf.at[0])
                    off = tb * NP
                    for j in range(CH // 16):
                        gidx[0, pl.ds(j * 16, 16)] = (
                            rowi[0, pl.ds(j * 16, 16)] + off)
                    pltpu.async_copy(hblk.at[gidx.at[0]], rows, sem).wait()
                    for g in range(CH // 16):
                        wv = wbuf[0, pl.ds(g * 16, 16)]
                        for l in range(16):
                            jj = g * 16 + l
                            wj = jnp.broadcast_to(
                                lax.slice(wv, (l,), (l + 1,)), (16,))
                            for cpart in range(FB // 16):
                                rows[jj, pl.ds(cpart * 16, 16)] = (
                                    rows[jj, pl.ds(cpart * 16, 16)] * wj)
                    pltpu.sync_copy(rows, acc_sp.at[coli.at[0]], add=True)
                return c
            lax.fori_loop(0, NIT, body, 0)
            plsc.subcore_barrier()

            def fcopy(kk, c):
                r0 = sid * STRIPE + kk * ZR
                pltpu.sync_copy(acc_sp.at[pl.ds(r0, ZR)], zrow)
                pltpu.sync_copy(zrow, out_hbm.at[pl.ds(tb * NP + r0, ZR)])
                return c
            lax.fori_loop(0, STRIPE // ZR, fcopy, 0)
            for r in range(ZR):
                for cpart in range(FB // 16):
                    zrow[r, pl.ds(cpart * 16, 16)] = zero16
            plsc.subcore_barrier()
            return carry
        lax.fori_loop(0, S, subpass, 0)

    return kern


def _gat_sc(h2d, row, col, nk16, a_src, a_dst, b, out_ch, concat):
    N = NUM_NODES
    S = out_ch // FB
    h3 = h2d.reshape(N, HEADS, out_ch)
    a_s = (h3 * a_src[None]).sum(-1)
    a_d = (h3 * a_dst[None]).sum(-1)
    asf = jnp.pad(a_s.T, ((0, 0), (0, NP - N))).reshape(HEADS * NP)
    adf = jnp.pad(a_d.T, ((0, 0), (0, NP - N))).reshape(HEADS * NP)
    hblk = (jnp.pad(h2d, ((0, NP - N), (0, 0)))
            .reshape(NP, HEADS * S, FB).transpose(1, 0, 2)
            .reshape(HEADS * S * NP, FB))
    E = row.shape[0]
    w_flat, asum_flat = _edge_w_kernel(E)(asf, adf, row, col, nk16)
    agg = _edge_agg_kernel(E, S)(hblk, row, col, w_flat, nk16)
    agg = (agg.reshape(HEADS, S, NP, FB)[:, :, :N, :]
           .transpose(2, 0, 1, 3).reshape(N, HEADS, out_ch))
    asum = asum_flat.reshape(HEADS, NP)[:, :N].T
    wself = jnp.exp(_leaky_relu(a_s + a_d))
    num = agg + wself[:, :, None] * h3
    den = asum + wself
    out = num / den[:, :, None]
    if concat:
        out = out.reshape(N, HEADS * out_ch)
    else:
        out = out.mean(axis=1)
    return out + b


def _instance_norm(x, batch, gamma, beta, num_graphs, eps=1e-5):
    ones = jnp.ones((x.shape[0],), x.dtype)
    cnt = jnp.maximum(jax.ops.segment_sum(ones, batch, num_segments=num_graphs), 1.0)[:, None]
    mean = jax.ops.segment_sum(x, batch, num_segments=num_graphs) / cnt
    var = jax.ops.segment_sum(x * x, batch, num_segments=num_graphs) / cnt - mean ** 2
    xh = (x - mean[batch]) * jax.lax.rsqrt(var[batch] + eps)
    return xh * gamma + beta


def _topk_perm(score, batch_key, valid, num_nodes):
    idx1 = jnp.argsort(-score)
    order = idx1[jnp.argsort(batch_key[idx1])]
    bs = batch_key[order]
    total = jax.ops.segment_sum(jnp.ones((num_nodes,), jnp.int32), batch_key, num_segments=NUM_GRAPHS + 1)
    starts = jnp.concatenate([jnp.zeros((1,), jnp.int32), jnp.cumsum(total)[:-1]])
    rank = jnp.arange(num_nodes, dtype=jnp.int32) - starts[bs]
    cnt = jax.ops.segment_sum(valid.astype(jnp.int32), batch_key, num_segments=NUM_GRAPHS + 1)
    k = jnp.maximum((3 * cnt + 9) // 10, 1)
    keep = valid[order] & (rank < k[bs])
    pos = jnp.cumsum(keep.astype(jnp.int32)) - 1
    nsel = keep.astype(jnp.int32).sum()
    src = jnp.zeros((num_nodes,), jnp.int32).at[jnp.where(keep, pos, num_nodes)].set(
        order.astype(jnp.int32), mode='drop')
    newidx = jnp.full((num_nodes,), -1, jnp.int32).at[order].set(jnp.where(keep, pos, -1))
    valid_new = jnp.arange(num_nodes, dtype=jnp.int32) < nsel
    return src, newidx, valid_new


def _filter_adj(row, col, newidx, pad_node, num_edges):
    nr = newidx[row]
    nc = newidx[col]
    keep = (nr >= 0) & (nc >= 0)
    pos = jnp.cumsum(keep.astype(jnp.int32)) - 1
    tgt = jnp.where(keep, pos, num_edges)
    row_new = jnp.full((num_edges,), pad_node, jnp.int32).at[tgt].set(nr, mode='drop')
    col_new = jnp.full((num_edges,), pad_node, jnp.int32).at[tgt].set(nc, mode='drop')
    nkept = jnp.sum(keep.astype(jnp.int32))
    return row_new, col_new, jnp.full((16,), 1, jnp.int32) * nkept


def kernel(x, params, edge_index, batch):
    p = params
    N0 = x.shape[0]
    pad_node = N0 - 1
    row0 = jnp.asarray(edge_index[0], jnp.int32)
    col0 = jnp.asarray(edge_index[1], jnp.int32)
    batch0 = jnp.asarray(batch, jnp.int32)
    nk_full = jnp.full((16,), NUM_EDGES, jnp.int32)
    h = _elu(_gat_sc(x @ p['W1'], row0, col0, nk_full, p['as1'], p['ad1'],
                     p['b1'], 32, True))
    h = _instance_norm(h, batch0, p['g1'], p['be1'], NUM_GRAPHS)
    score1 = jax.nn.sigmoid((h @ p['pw1']) / (jnp.linalg.norm(p['pw1']) + 1e-16))
    valid0 = jnp.ones((N0,), bool)
    src1, newidx1, valid1 = _topk_perm(score1, batch0, valid0, N0)
    row1, col1, nk1 = _filter_adj(row0, col0, newidx1, pad_node, NUM_EDGES)
    h = jnp.where(valid1[:, None], h[src1] * score1[src1][:, None], 0.0)
    batch1 = jnp.where(valid1, batch0[src1], NUM_GRAPHS).astype(jnp.int32)
    h = _elu(_gat_sc(h @ p['W2'], row1, col1, nk1, p['as2'], p['ad2'],
                     p['b2'], 128, True))
    h = _instance_norm(h, batch1, p['g2'], p['be2'], NUM_GRAPHS)
    h = jnp.where(valid1[:, None], h, 0.0)
    score2 = jax.nn.sigmoid((h @ p['pw2']) / (jnp.linalg.norm(p['pw2']) + 1e-16))
    src2, newidx2, valid2 = _topk_perm(score2, batch1, valid1, N0)
    row2, col2, nk2 = _filter_adj(row1, col1, newidx2, pad_node, NUM_EDGES)
    h = jnp.where(valid2[:, None], h[src2] * score2[src2][:, None], 0.0)
    batch2 = jnp.where(valid2, batch1[src2], NUM_GRAPHS).astype(jnp.int32)
    h = _elu(_gat_sc(h @ p['W3'], row2, col2, nk2, p['as3'], p['ad3'],
                     p['b3'], 512, False))
    h = _instance_norm(h, batch2, p['g3'], p['be3'], NUM_GRAPHS)
    h = jnp.where(valid2[:, None], h, 0.0)
    gate = _elu(h @ p['Gw1'] + p['Gb1'])
    gate = _elu(gate @ p['Gw2'] + p['Gb2'])
    gate = (gate @ p['Gw3'] + p['Gb3'])[:, 0]
    feat = _elu(h @ p['Aw'] + p['Ab'])
    gmax = jax.ops.segment_max(gate, batch2, num_segments=NUM_GRAPHS)
    gmax = jnp.where(jnp.isfinite(gmax), gmax, 0.0)
    ge = jnp.exp(gate - gmax[batch2])
    gs = jax.ops.segment_sum(ge, batch2, num_segments=NUM_GRAPHS)
    w = ge / (gs[batch2] + 1e-16)
    pooled = jax.ops.segment_sum(feat * w[:, None], batch2, num_segments=NUM_GRAPHS)
    return _head(pooled, p)
